# baseline (speedup 1.0000x reference)
import functools

import jax
import jax.numpy as jnp
from jax import lax
from jax.experimental import pallas as pl
from jax.experimental.pallas import tpu as pltpu
from jax.experimental.pallas import tpu_sc as plsc

H = 768
F = 2048
E = 64
N = 4096
B_TOK = 256
NB = N // B_TOK
G = NB + E - 1

NW = 32
BPW = N // NW


def _router_body(x_ref, wr_ref, out_ref, cnt_ref):
    t = pl.program_id(0)
    logits = lax.dot_general(
        x_ref[...], wr_ref[...], (((1,), (1,)), ((), ())),
        preferred_element_type=jnp.float32)
    m = jnp.max(logits, axis=1, keepdims=True)
    col = lax.broadcasted_iota(jnp.int32, logits.shape, 1)
    eid = jnp.min(jnp.where(logits >= m, col, E), axis=1).astype(jnp.int32)
    out_ref[...] = eid.reshape(out_ref.shape)

    @pl.when(t == 0)
    def _():
        cnt_ref[...] = jnp.zeros_like(cnt_ref)

    onehot = (col == eid[:, None]).astype(jnp.int32)
    cnt_ref[...] += jnp.sum(onehot, axis=0, keepdims=True)


def _router(x_flat, Wr):
    out, cnt = pl.pallas_call(
        _router_body,
        grid=(8,),
        in_specs=[
            pl.BlockSpec((512, H), lambda t: (t, 0)),
            pl.BlockSpec((E, H), lambda t: (0, 0)),
        ],
        out_specs=[
            pl.BlockSpec((1, 4, 128), lambda t: (t, 0, 0)),
            pl.BlockSpec((1, E), lambda t: (0, 0)),
        ],
        out_shape=[
            jax.ShapeDtypeStruct((8, 4, 128), jnp.int32),
            jax.ShapeDtypeStruct((1, E), jnp.int32),
        ],
        compiler_params=pltpu.CompilerParams(
            dimension_semantics=("arbitrary",)),
    )(x_flat, Wr)
    return out.reshape(N), cnt.reshape(E)


@functools.lru_cache(maxsize=None)
def _sc_gather_kernel():
    mesh = plsc.VectorSubcoreMesh(core_axis_name="c", subcore_axis_name="s")

    @functools.partial(
        pl.kernel,
        mesh=mesh,
        out_type=jax.ShapeDtypeStruct((N, H), jnp.float32),
        scratch_types=[
            pltpu.VMEM((BPW,), jnp.int32),
            pltpu.VMEM((BPW, H), jnp.float32),
            pltpu.SemaphoreType.DMA,
        ],
    )
    def gather(x_hbm, idx_hbm, out_hbm, idx_v, rows_v, sem):
        wid = lax.axis_index("s") * 2 + lax.axis_index("c")
        base = wid * BPW
        pltpu.sync_copy(idx_hbm.at[pl.ds(base, BPW)], idx_v)
        pltpu.async_copy(x_hbm.at[idx_v], rows_v, sem).wait()
        pltpu.sync_copy(rows_v, out_hbm.at[pl.ds(base, BPW)])

    return gather


@functools.lru_cache(maxsize=None)
def _sc_scatter_kernel():
    mesh = plsc.VectorSubcoreMesh(core_axis_name="c", subcore_axis_name="s")

    @functools.partial(
        pl.kernel,
        mesh=mesh,
        out_type=jax.ShapeDtypeStruct((N, H), jnp.float32),
        scratch_types=[
            pltpu.VMEM((BPW,), jnp.int32),
            pltpu.VMEM((BPW, H), jnp.float32),
            pltpu.SemaphoreType.DMA,
        ],
    )
    def scatter(y_hbm, idx_hbm, out_hbm, idx_v, rows_v, sem):
        wid = lax.axis_index("s") * 2 + lax.axis_index("c")
        base = wid * BPW
        pltpu.sync_copy(idx_hbm.at[pl.ds(base, BPW)], idx_v)
        pltpu.sync_copy(y_hbm.at[pl.ds(base, BPW)], rows_v)
        pltpu.async_copy(rows_v, out_hbm.at[idx_v], sem).wait()

    return scatter


def _gmm_body(starts_ref, ends_ref, fb_ref, nb_ref,
              x_hbm, wg_hbm, wu_hbm, wd_hbm, out_ref,
              xv, wgv, wuv, wdv, yb, st, xsem, wsem, fsem):

    def issue_w(e, slot):
        pltpu.make_async_copy(wg_hbm.at[e], wgv.at[slot], wsem.at[slot]).start()
        pltpu.make_async_copy(wu_hbm.at[e], wuv.at[slot], wsem.at[slot]).start()
        pltpu.make_async_copy(wd_hbm.at[e], wdv.at[slot], wsem.at[slot]).start()

    def wait_w(slot):
        pltpu.make_async_copy(wg_hbm.at[0], wgv.at[slot], wsem.at[slot]).wait()
        pltpu.make_async_copy(wu_hbm.at[0], wuv.at[slot], wsem.at[slot]).wait()
        pltpu.make_async_copy(wd_hbm.at[0], wdv.at[slot], wsem.at[slot]).wait()

    issue_w(0, 0)
    issue_w(1, 1)
    pltpu.make_async_copy(x_hbm, xv, xsem).start()
    st[0] = -1
    st[1] = 0
    st[2] = 0
    st[3] = 0
    pltpu.make_async_copy(x_hbm, xv, xsem).wait()

    def expert_step(e, carry):
        slot = lax.rem(e, 2)
        wait_w(slot)
        s_e = starts_ref[e]
        e_e = ends_ref[e]
        f_e = fb_ref[e]

        def chunk(k, c2):
            t = f_e + k
            base = t * B_TOK
            lo = jnp.maximum(s_e - base, 0)
            hi = jnp.minimum(e_e - base, B_TOK)
            rows = lax.broadcasted_iota(jnp.int32, (B_TOK, 1), 0)
            msk = ((rows >= lo) & (rows < hi)).astype(jnp.float32)
            xm = xv[pl.ds(base, B_TOK), :] * msk
            g = lax.dot_general(xm, wgv[slot], (((1,), (1,)), ((), ())),
                                preferred_element_type=jnp.float32)
            u = lax.dot_general(xm, wuv[slot], (((1,), (1,)), ((), ())),
                                preferred_element_type=jnp.float32)
            hdn = (g * jax.nn.sigmoid(g)) * u
            y = lax.dot_general(hdn, wdv[slot], (((1,), (1,)), ((), ())),
                                preferred_element_type=jnp.float32)
            cur = st[0]
            oslot = st[1]

            @pl.when(t == cur)
            def _():
                yb[oslot] += y

            @pl.when(t != cur)
            def _():
                @pl.when(cur >= 0)
                def _():
                    pltpu.make_async_copy(
                        yb.at[oslot],
                        out_ref.at[pl.ds(cur * B_TOK, B_TOK), :],
                        fsem.at[oslot]).start()
                    st[2 + oslot] = 1

                ns = 1 - oslot

                @pl.when(st[2 + ns] == 1)
                def _():
                    pltpu.make_async_copy(
                        out_ref.at[pl.ds(0, B_TOK), :],
                        yb.at[ns],
                        fsem.at[ns]).wait()
                    st[2 + ns] = 0

                yb[ns] = y
                st[0] = t
                st[1] = ns

            return c2

        lax.fori_loop(0, nb_ref[e], chunk, 0)

        @pl.when(e + 2 < E)
        def _():
            issue_w(e + 2, slot)

        return carry

    lax.fori_loop(0, E, expert_step, 0)

    cur = st[0]
    slot = st[1]
    pltpu.make_async_copy(
        yb.at[slot],
        out_ref.at[pl.ds(cur * B_TOK, B_TOK), :],
        fsem.at[slot]).start()
    st[2 + slot] = 1
    for s in (0, 1):
        @pl.when(st[2 + s] == 1)
        def _():
            pltpu.make_async_copy(
                out_ref.at[pl.ds(0, B_TOK), :],
                yb.at[s],
                fsem.at[s]).wait()


def _gmm(x_sorted, Wg, Wu, Wd, starts, ends, fb, nb):
    grid_spec = pltpu.PrefetchScalarGridSpec(
        num_scalar_prefetch=4,
        grid=(1,),
        in_specs=[
            pl.BlockSpec(memory_space=pl.ANY),
            pl.BlockSpec(memory_space=pl.ANY),
            pl.BlockSpec(memory_space=pl.ANY),
            pl.BlockSpec(memory_space=pl.ANY),
        ],
        out_specs=pl.BlockSpec(memory_space=pl.ANY),
        scratch_shapes=[
            pltpu.VMEM((N, H), jnp.float32),
            pltpu.VMEM((2, F, H), jnp.float32),
            pltpu.VMEM((2, F, H), jnp.float32),
            pltpu.VMEM((2, H, F), jnp.float32),
            pltpu.VMEM((2, B_TOK, H), jnp.float32),
            pltpu.SMEM((4,), jnp.int32),
            pltpu.SemaphoreType.DMA,
            pltpu.SemaphoreType.DMA((2,)),
            pltpu.SemaphoreType.DMA((2,)),
        ],
    )
    return pl.pallas_call(
        _gmm_body,
        grid_spec=grid_spec,
        out_shape=jax.ShapeDtypeStruct((N, H), jnp.float32),
        compiler_params=pltpu.CompilerParams(
            dimension_semantics=("arbitrary",)),
    )(starts, ends, fb, nb, x_sorted, Wg, Wu, Wd)


def _schedule(counts):
    ends = jnp.cumsum(counts)
    starts = ends - counts
    nonempty = counts > 0
    fb = jnp.where(nonempty, starts // B_TOK, 0).astype(jnp.int32)
    lb = jnp.where(nonempty, (ends - 1) // B_TOK, -1)
    nb = jnp.where(nonempty, lb - fb + 1, 0).astype(jnp.int32)
    return starts.astype(jnp.int32), ends.astype(jnp.int32), fb, nb


def kernel(x, Wr, Wg, Wu, Wd):
    b, s, d = x.shape
    x_flat = x.reshape(N, H)
    eid, counts = _router(x_flat, Wr)
    sort_idx = jnp.argsort(eid).astype(jnp.int32)
    starts, ends, fb, nb = _schedule(counts)
    x_sorted = _sc_gather_kernel()(x_flat, sort_idx)
    y_sorted = _gmm(x_sorted, Wg, Wu, Wd, starts, ends, fb, nb)
    out = _sc_scatter_kernel()(y_sorted, sort_idx)
    return out.reshape(b, s, d)


# speedup vs baseline: 1.0067x; 1.0067x over previous
import functools

import jax
import jax.numpy as jnp
from jax import lax
from jax.experimental import pallas as pl
from jax.experimental.pallas import tpu as pltpu
from jax.experimental.pallas import tpu_sc as plsc

H = 768
F = 2048
E = 64
N = 4096
B_TOK = 256
NB = N // B_TOK
G = NB + E - 1

NW = 32
BPW = N // NW


def _router_body(x_ref, wr_ref, out_ref, cnt_ref):
    t = pl.program_id(0)
    logits = lax.dot_general(
        x_ref[...], wr_ref[...], (((1,), (1,)), ((), ())),
        preferred_element_type=jnp.float32)
    m = jnp.max(logits, axis=1, keepdims=True)
    col = lax.broadcasted_iota(jnp.int32, logits.shape, 1)
    eid = jnp.min(jnp.where(logits >= m, col, E), axis=1).astype(jnp.int32)
    out_ref[...] = eid.reshape(out_ref.shape)

    @pl.when(t == 0)
    def _():
        cnt_ref[...] = jnp.zeros_like(cnt_ref)

    onehot = (col == eid[:, None]).astype(jnp.int32)
    cnt_ref[...] += jnp.sum(onehot, axis=0, keepdims=True)


def _router(x_flat, Wr):
    out, cnt = pl.pallas_call(
        _router_body,
        grid=(8,),
        in_specs=[
            pl.BlockSpec((512, H), lambda t: (t, 0)),
            pl.BlockSpec((E, H), lambda t: (0, 0)),
        ],
        out_specs=[
            pl.BlockSpec((1, 4, 128), lambda t: (t, 0, 0)),
            pl.BlockSpec((1, E), lambda t: (0, 0)),
        ],
        out_shape=[
            jax.ShapeDtypeStruct((8, 4, 128), jnp.int32),
            jax.ShapeDtypeStruct((1, E), jnp.int32),
        ],
        compiler_params=pltpu.CompilerParams(
            dimension_semantics=("arbitrary",)),
    )(x_flat, Wr)
    return out.reshape(N), cnt.reshape(E)


@functools.lru_cache(maxsize=None)
def _sc_gather_kernel():
    mesh = plsc.VectorSubcoreMesh(core_axis_name="c", subcore_axis_name="s")

    @functools.partial(
        pl.kernel,
        mesh=mesh,
        out_type=jax.ShapeDtypeStruct((N, H), jnp.float32),
        scratch_types=[
            pltpu.VMEM((BPW,), jnp.int32),
            pltpu.VMEM((BPW, H), jnp.float32),
            pltpu.SemaphoreType.DMA,
        ],
    )
    def gather(x_hbm, idx_hbm, out_hbm, idx_v, rows_v, sem):
        wid = lax.axis_index("s") * 2 + lax.axis_index("c")
        base = wid * BPW
        pltpu.sync_copy(idx_hbm.at[pl.ds(base, BPW)], idx_v)
        pltpu.async_copy(x_hbm.at[idx_v], rows_v, sem).wait()
        pltpu.sync_copy(rows_v, out_hbm.at[pl.ds(base, BPW)])

    return gather


@functools.lru_cache(maxsize=None)
def _sc_scatter_kernel():
    mesh = plsc.VectorSubcoreMesh(core_axis_name="c", subcore_axis_name="s")

    @functools.partial(
        pl.kernel,
        mesh=mesh,
        out_type=jax.ShapeDtypeStruct((N, H), jnp.float32),
        scratch_types=[
            pltpu.VMEM((BPW,), jnp.int32),
            pltpu.VMEM((BPW, H), jnp.float32),
            pltpu.SemaphoreType.DMA,
        ],
    )
    def scatter(y_hbm, idx_hbm, out_hbm, idx_v, rows_v, sem):
        wid = lax.axis_index("s") * 2 + lax.axis_index("c")
        base = wid * BPW
        pltpu.sync_copy(idx_hbm.at[pl.ds(base, BPW)], idx_v)
        pltpu.sync_copy(y_hbm.at[pl.ds(base, BPW)], rows_v)
        pltpu.async_copy(rows_v, out_hbm.at[idx_v], sem).wait()

    return scatter


def _gmm_body(starts_ref, ends_ref, fb_ref, nb_ref,
              x_ref, wg_ref, wu_ref, wd_ref, out_ref, yb, st, sems):
    e = pl.program_id(0)

    @pl.when(e == 0)
    def _():
        st[0] = -1
        st[1] = 0
        st[2] = 0
        st[3] = 0

    s_e = starts_ref[e]
    e_e = ends_ref[e]
    f_e = fb_ref[e]

    def chunk(k, carry):
        t = f_e + k
        base = t * B_TOK
        lo = jnp.maximum(s_e - base, 0)
        hi = jnp.minimum(e_e - base, B_TOK)
        rows = lax.broadcasted_iota(jnp.int32, (B_TOK, 1), 0)
        msk = ((rows >= lo) & (rows < hi)).astype(jnp.float32)
        xm = x_ref[pl.ds(base, B_TOK), :] * msk
        g = lax.dot_general(xm, wg_ref[0], (((1,), (1,)), ((), ())),
                            preferred_element_type=jnp.float32)
        u = lax.dot_general(xm, wu_ref[0], (((1,), (1,)), ((), ())),
                            preferred_element_type=jnp.float32)
        hdn = (g * jax.nn.sigmoid(g)) * u
        y = lax.dot_general(hdn, wd_ref[0], (((1,), (1,)), ((), ())),
                            preferred_element_type=jnp.float32)
        cur = st[0]
        slot = st[1]

        @pl.when(t == cur)
        def _():
            yb[slot] += y

        @pl.when(t != cur)
        def _():
            @pl.when(cur >= 0)
            def _():
                pltpu.make_async_copy(
                    yb.at[slot],
                    out_ref.at[pl.ds(cur * B_TOK, B_TOK), :],
                    sems.at[slot]).start()
                st[2 + slot] = 1

            ns = 1 - slot

            @pl.when(st[2 + ns] == 1)
            def _():
                pltpu.make_async_copy(
                    out_ref.at[pl.ds(0, B_TOK), :],
                    yb.at[ns],
                    sems.at[ns]).wait()
                st[2 + ns] = 0

            yb[ns] = y
            st[0] = t
            st[1] = ns

        return carry

    lax.fori_loop(0, nb_ref[e], chunk, 0)

    @pl.when(e == E - 1)
    def _():
        cur = st[0]
        slot = st[1]
        pltpu.make_async_copy(
            yb.at[slot],
            out_ref.at[pl.ds(cur * B_TOK, B_TOK), :],
            sems.at[slot]).start()
        st[2 + slot] = 1
        for s in (0, 1):
            @pl.when(st[2 + s] == 1)
            def _():
                pltpu.make_async_copy(
                    out_ref.at[pl.ds(0, B_TOK), :],
                    yb.at[s],
                    sems.at[s]).wait()


def _gmm(x_sorted, Wg, Wu, Wd, starts, ends, fb, nb):
    grid_spec = pltpu.PrefetchScalarGridSpec(
        num_scalar_prefetch=4,
        grid=(E,),
        in_specs=[
            pl.BlockSpec((N, H), lambda e, *s: (0, 0)),
            pl.BlockSpec((1, F, H), lambda e, *s: (e, 0, 0)),
            pl.BlockSpec((1, F, H), lambda e, *s: (e, 0, 0)),
            pl.BlockSpec((1, H, F), lambda e, *s: (e, 0, 0)),
        ],
        out_specs=pl.BlockSpec(memory_space=pl.ANY),
        scratch_shapes=[
            pltpu.VMEM((2, B_TOK, H), jnp.float32),
            pltpu.SMEM((4,), jnp.int32),
            pltpu.SemaphoreType.DMA((2,)),
        ],
    )
    return pl.pallas_call(
        _gmm_body,
        grid_spec=grid_spec,
        out_shape=jax.ShapeDtypeStruct((N, H), jnp.float32),
        compiler_params=pltpu.CompilerParams(
            dimension_semantics=("arbitrary",)),
    )(starts, ends, fb, nb, x_sorted, Wg, Wu, Wd)


def _schedule(counts):
    ends = jnp.cumsum(counts)
    starts = ends - counts
    nonempty = counts > 0
    fb = jnp.where(nonempty, starts // B_TOK, 0).astype(jnp.int32)
    lb = jnp.where(nonempty, (ends - 1) // B_TOK, -1)
    nb = jnp.where(nonempty, lb - fb + 1, 0).astype(jnp.int32)
    return starts.astype(jnp.int32), ends.astype(jnp.int32), fb, nb


def kernel(x, Wr, Wg, Wu, Wd):
    b, s, d = x.shape
    x_flat = x.reshape(N, H)
    eid, counts = _router(x_flat, Wr)
    sort_idx = jnp.argsort(eid).astype(jnp.int32)
    starts, ends, fb, nb = _schedule(counts)
    x_sorted = _sc_gather_kernel()(x_flat, sort_idx)
    y_sorted = _gmm(x_sorted, Wg, Wu, Wd, starts, ends, fb, nb)
    out = _sc_scatter_kernel()(y_sorted, sort_idx)
    return out.reshape(b, s, d)
